# baseline (speedup 1.0000x reference)
import jax
import jax.numpy as jnp
from jax import lax
from jax.experimental import pallas as pl
from jax.experimental.pallas import tpu as pltpu
from jax.experimental.pallas import tpu_sc as plsc

T, N, E, D, H, R = 4, 10000, 320000, 128, 128, 16
NC, NS = 2, 16
NW = NC * NS
EPW = E // NW
BB = 80
NBLK = EPW // BB
EPS = E // NC
ZCH = 40
ZTW = 10
ZPT = N // ZTW
LG = H // 16
W32 = H // 2


TW = T * W32


def _gather(p_tab, q_tab, ridx, cidx, ne, bb):
    epw = ne // NW
    nblk = epw // bb

    def body(p_tab, q_tab, ridx, cidx, p_out, q_out,
             idx_r, idx_c, bp0, bq0, bp1, bq1, sem0, sem1):
        cid = lax.axis_index("c")
        sid = lax.axis_index("s")
        wid = sid * NC + cid

        def fire(bp, bq, sem, eb):
            pltpu.async_copy(p_tab.at[idx_r.at[pl.ds(eb, bb)]], bp, sem)
            pltpu.async_copy(q_tab.at[idx_c.at[pl.ds(eb, bb)]], bq, sem)

        def drain(bp, bq, sem):
            pltpu.make_async_copy(p_tab.at[idx_r.at[pl.ds(0, bb)]],
                                  bp, sem).wait()
            pltpu.make_async_copy(q_tab.at[idx_c.at[pl.ds(0, bb)]],
                                  bq, sem).wait()

        def store(bp, bq, base_out):
            pltpu.sync_copy(bp, p_out.at[pl.ds(base_out, bb)])
            pltpu.sync_copy(bq, q_out.at[pl.ds(base_out, bb)])

        base_e = wid * epw
        pltpu.sync_copy(ridx.at[pl.ds(base_e, epw)], idx_r)
        pltpu.sync_copy(cidx.at[pl.ds(base_e, epw)], idx_c)
        fire(bp0, bq0, sem0, 0)

        def pair(k, c):
            eb0 = 2 * k * bb
            fire(bp1, bq1, sem1, eb0 + bb)
            drain(bp0, bq0, sem0)
            store(bp0, bq0, base_e + eb0)
            fire(bp0, bq0, sem0, eb0 + 2 * bb)
            drain(bp1, bq1, sem1)
            store(bp1, bq1, base_e + eb0 + bb)
            return c

        lax.fori_loop(0, (nblk - 1) // 2, pair, 0)
        drain(bp0, bq0, sem0)
        store(bp0, bq0, base_e + (nblk - 1) * bb)

    return pl.kernel(
        body,
        out_type=(jax.ShapeDtypeStruct((ne, TW), jnp.int32),
                  jax.ShapeDtypeStruct((ne, TW), jnp.int32)),
        mesh=plsc.VectorSubcoreMesh(core_axis_name="c", subcore_axis_name="s"),
        scratch_types=[
            pltpu.VMEM((epw,), jnp.int32),
            pltpu.VMEM((epw,), jnp.int32),
            pltpu.VMEM((bb, TW), jnp.int32),
            pltpu.VMEM((bb, TW), jnp.int32),
            pltpu.VMEM((bb, TW), jnp.int32),
            pltpu.VMEM((bb, TW), jnp.int32),
            pltpu.SemaphoreType.DMA,
            pltpu.SemaphoreType.DMA,
        ],
    )(p_tab, q_tab, ridx, cidx)


def _scatter(e2, row3d, ne, bb):
    epw = ne // NW
    nblk = epw // bb
    eps = ne // NC

    def body(e2, row3d, parts, idx2d, buf, buf1, zbuf, acc, sem0, sem1):
        cid = lax.axis_index("c")
        sid = lax.axis_index("s")
        w = cid * NS + sid
        pltpu.sync_copy(row3d.at[w], idx2d)

        def zrow(r, c):
            for j in range(LG):
                zbuf[r, pl.ds(j * 16, 16)] = jnp.zeros((16,), jnp.float32)
            return c

        lax.fori_loop(0, ZCH, zrow, 0)

        tbase = cid * eps + sid * epw

        def fire(b, bf, sem, t):
            pltpu.async_copy(e2.at[t, pl.ds(tbase + b * bb, bb)], bf, sem)

        def drain(bf, sem, t):
            pltpu.make_async_copy(e2.at[t, pl.ds(tbase, bb)], bf, sem).wait()

        for t in range(T):
            @pl.when(sid < ZTW)
            def _zero():
                def zc(k, c):
                    pltpu.sync_copy(zbuf,
                                    acc.at[pl.ds(sid * ZPT + k * ZCH, ZCH)])
                    return c
                lax.fori_loop(0, ZPT // ZCH, zc, 0)

            plsc.subcore_barrier()
            fire(0, buf, sem0, t)

            def pair(k, c):
                b0 = 2 * k
                fire(b0 + 1, buf1, sem1, t)
                drain(buf, sem0, t)
                pltpu.sync_copy(buf, acc.at[idx2d.at[b0]], add=True)
                fire(b0 + 2, buf, sem0, t)
                drain(buf1, sem1, t)
                pltpu.sync_copy(buf1, acc.at[idx2d.at[b0 + 1]], add=True)
                return c

            lax.fori_loop(0, (nblk - 1) // 2, pair, 0)
            drain(buf, sem0, t)
            pltpu.sync_copy(buf, acc.at[idx2d.at[nblk - 1]], add=True)
            plsc.subcore_barrier()

            @pl.when(sid < ZTW)
            def _readback():
                def rb(k, c):
                    off = sid * ZPT + k * ZCH
                    pltpu.sync_copy(acc.at[pl.ds(off, ZCH)],
                                    parts.at[t, cid, pl.ds(off, ZCH)])
                    return c
                lax.fori_loop(0, ZPT // ZCH, rb, 0)

            plsc.subcore_barrier()

    return pl.kernel(
        body,
        out_type=jax.ShapeDtypeStruct((T, NC, N, H), jnp.float32),
        mesh=plsc.VectorSubcoreMesh(core_axis_name="c", subcore_axis_name="s"),
        scratch_types=[
            pltpu.VMEM((nblk, bb), jnp.int32),
            pltpu.VMEM((bb, H), jnp.float32),
            pltpu.VMEM((bb, H), jnp.float32),
            pltpu.VMEM((ZCH, H), jnp.float32),
            pltpu.VMEM_SHARED((N, H), jnp.float32),
            pltpu.SemaphoreType.DMA,
            pltpu.SemaphoreType.DMA,
        ],
    )(e2, row3d)


BA = 2000


def _pack_bf16_pairs(x):
    xi = lax.bitcast_convert_type(x, jnp.int32)
    rne = xi + 0x7FFF + (lax.shift_right_logical(xi, 16) & 1)
    xb = lax.shift_right_logical(rne, 16)
    return xb[:, :W32] | (xb[:, W32:] << 16)


def _tables_tc(h_ref, wa_ref, wb_ref, p_ref, q_ref):
    for t in range(T):
        hb = h_ref[t]
        pf = jnp.dot(hb, wa_ref[...], preferred_element_type=jnp.float32)
        qf = jnp.dot(hb, wb_ref[...], preferred_element_type=jnp.float32)
        p_ref[:, t * W32:(t + 1) * W32] = _pack_bf16_pairs(pf)
        q_ref[:, t * W32:(t + 1) * W32] = _pack_bf16_pairs(qf)


def _tables(h, wa, wb):
    return pl.pallas_call(
        _tables_tc,
        grid=(N // BA,),
        in_specs=[
            pl.BlockSpec((T, BA, D), lambda j: (0, j, 0)),
            pl.BlockSpec((D, H), lambda j: (0, 0)),
            pl.BlockSpec((D, H), lambda j: (0, 0)),
        ],
        out_specs=[pl.BlockSpec((BA, TW), lambda j: (j, 0))] * 2,
        out_shape=[jax.ShapeDtypeStruct((N, TW), jnp.int32)] * 2,
    )(h, wa, wb)


BC = 2000


def _unpack_sum(pw, qw):
    plo = lax.bitcast_convert_type(pw << 16, jnp.float32)
    phi = lax.bitcast_convert_type(pw & jnp.int32(-65536), jnp.float32)
    qlo = lax.bitcast_convert_type(qw << 16, jnp.float32)
    qhi = lax.bitcast_convert_type(qw & jnp.int32(-65536), jnp.float32)
    return jnp.concatenate([plo + qlo, phi + qhi], axis=1)


def _edge_tc(p_ref, q_ref, rad_ref, wc_ref, b1_ref, w2_ref, b2_ref, e2_ref):
    p = p_ref[...]
    q = q_ref[...]
    for t in range(T):
        g = _unpack_sum(p[:, t * W32:(t + 1) * W32],
                        q[:, t * W32:(t + 1) * W32])
        e1 = (g
              + jnp.dot(rad_ref[t], wc_ref[...],
                        preferred_element_type=jnp.float32)
              + b1_ref[...])
        e1 = jnp.maximum(e1, 0.0)
        e2 = jnp.dot(e1, w2_ref[...], preferred_element_type=jnp.float32)
        e2_ref[t] = jnp.maximum(e2 + b2_ref[...], 0.0)


def _edge_mlp(p_rows, q_rows, radial, wc, b1, w2, b2, ne, off):
    return pl.pallas_call(
        _edge_tc,
        grid=(ne // BC,),
        in_specs=[
            pl.BlockSpec((BC, TW), lambda i: (i, 0)),
            pl.BlockSpec((BC, TW), lambda i: (i, 0)),
            pl.BlockSpec((T, BC, R), lambda i, off=off: (0, i + off, 0)),
            pl.BlockSpec((R, H), lambda i: (0, 0)),
            pl.BlockSpec((1, H), lambda i: (0, 0)),
            pl.BlockSpec((H, H), lambda i: (0, 0)),
            pl.BlockSpec((1, H), lambda i: (0, 0)),
        ],
        out_specs=pl.BlockSpec((T, BC, H), lambda i: (0, i, 0)),
        out_shape=jax.ShapeDtypeStruct((T, ne, H), jnp.float32),
    )(p_rows, q_rows, radial, wc, b1, w2, b2)


BN = 2000


def _node_tc(oth_ref, h_ref, pa_ref, pb_ref, wn1_ref, bn1_ref, wn2_ref,
             bn2_ref, hout_ref, a_ref):
    oth = oth_ref[0]
    hb = h_ref[0]
    agg = (pa_ref[0, 0] + pa_ref[0, 1]) + (pb_ref[0, 0] + pb_ref[0, 1])
    a = jnp.concatenate([oth, hb, agg], axis=1)
    a_ref[0] = a
    z = jnp.maximum(
        jnp.dot(a, wn1_ref[...], preferred_element_type=jnp.float32)
        + bn1_ref[...], 0.0)
    o = jnp.dot(z, wn2_ref[...], preferred_element_type=jnp.float32) \
        + bn2_ref[...]
    hout_ref[0] = hb + o


def _node_mlp(others, h, parts_a, parts_b, wn1, bn1, wn2, bn2):
    return pl.pallas_call(
        _node_tc,
        grid=(T, N // BN),
        in_specs=[
            pl.BlockSpec((1, BN, H), lambda t, j: (t, j, 0)),
            pl.BlockSpec((1, BN, D), lambda t, j: (t, j, 0)),
            pl.BlockSpec((1, NC, BN, H), lambda t, j: (t, 0, j, 0)),
            pl.BlockSpec((1, NC, BN, H), lambda t, j: (t, 0, j, 0)),
            pl.BlockSpec((H + D + H, H), lambda t, j: (0, 0)),
            pl.BlockSpec((1, H), lambda t, j: (0, 0)),
            pl.BlockSpec((H, D), lambda t, j: (0, 0)),
            pl.BlockSpec((1, D), lambda t, j: (0, 0)),
        ],
        out_specs=[
            pl.BlockSpec((1, BN, D), lambda t, j: (t, j, 0)),
            pl.BlockSpec((1, BN, H + D + H), lambda t, j: (t, j, 0)),
        ],
        out_shape=[
            jax.ShapeDtypeStruct((T, N, D), jnp.float32),
            jax.ShapeDtypeStruct((T, N, H + D + H), jnp.float32),
        ],
    )(others, h, parts_a, parts_b, wn1, bn1, wn2, bn2)


def kernel(h, edge_index, radial, others,
           We1, be1, We2, be2, Wn1, bn1, Wn2, bn2):
    row = edge_index[0]
    col = edge_index[1]

    p32, q32 = _tables(h, We1[:D], We1[D:2 * D])

    half = E // 2
    bbc = 40
    parts = []
    for k in range(2):
        rk = row[k * half:(k + 1) * half]
        ck = col[k * half:(k + 1) * half]
        pr, qr = _gather(p32, q32, rk, ck, half, bbc)
        e2k = _edge_mlp(pr, qr, radial, We1[2 * D:], be1.reshape(1, H),
                        We2, be2.reshape(1, H), half, k * (half // BC))
        parts.append(_scatter(e2k, rk.reshape(NW, half // NW // bbc, bbc),
                              half, bbc))

    h_out, a_out = _node_mlp(others, h, parts[0], parts[1], Wn1,
                             bn1.reshape(1, H), Wn2, bn2.reshape(1, D))
    return h_out, a_out


# speedup vs baseline: 1.0538x; 1.0538x over previous
import jax
import jax.numpy as jnp
from jax import lax
from jax.experimental import pallas as pl
from jax.experimental.pallas import tpu as pltpu
from jax.experimental.pallas import tpu_sc as plsc

T, N, E, D, H, R = 4, 10000, 320000, 128, 128, 16
NC, NS = 2, 16
NW = NC * NS
EPW = E // NW
BB = 80
NBLK = EPW // BB
EPS = E // NC
ZCH = 40
ZTW = 10
ZPT = N // ZTW
LG = H // 16
W32 = H // 2


TW = T * W32


def _gather(p_tab, q_tab, ridx, cidx, ne, bb):
    epw = ne // NW
    nblk = epw // bb

    def body(p_tab, q_tab, ridx, cidx, p_out, q_out,
             idx_r, idx_c, bp0, bq0, bp1, bq1, sem0, sem1):
        cid = lax.axis_index("c")
        sid = lax.axis_index("s")
        wid = sid * NC + cid

        def fire(bp, bq, sem, eb):
            pltpu.async_copy(p_tab.at[idx_r.at[pl.ds(eb, bb)]], bp, sem)
            pltpu.async_copy(q_tab.at[idx_c.at[pl.ds(eb, bb)]], bq, sem)

        def drain(bp, bq, sem):
            pltpu.make_async_copy(p_tab.at[idx_r.at[pl.ds(0, bb)]],
                                  bp, sem).wait()
            pltpu.make_async_copy(q_tab.at[idx_c.at[pl.ds(0, bb)]],
                                  bq, sem).wait()

        def store(bp, bq, base_out):
            pltpu.sync_copy(bp, p_out.at[pl.ds(base_out, bb)])
            pltpu.sync_copy(bq, q_out.at[pl.ds(base_out, bb)])

        base_e = wid * epw
        pltpu.sync_copy(ridx.at[pl.ds(base_e, epw)], idx_r)
        pltpu.sync_copy(cidx.at[pl.ds(base_e, epw)], idx_c)
        fire(bp0, bq0, sem0, 0)

        def pair(k, c):
            eb0 = 2 * k * bb
            fire(bp1, bq1, sem1, eb0 + bb)
            drain(bp0, bq0, sem0)
            store(bp0, bq0, base_e + eb0)
            fire(bp0, bq0, sem0, eb0 + 2 * bb)
            drain(bp1, bq1, sem1)
            store(bp1, bq1, base_e + eb0 + bb)
            return c

        lax.fori_loop(0, (nblk - 1) // 2, pair, 0)
        drain(bp0, bq0, sem0)
        store(bp0, bq0, base_e + (nblk - 1) * bb)

    return pl.kernel(
        body,
        out_type=(jax.ShapeDtypeStruct((ne, TW), jnp.int32),
                  jax.ShapeDtypeStruct((ne, TW), jnp.int32)),
        mesh=plsc.VectorSubcoreMesh(core_axis_name="c", subcore_axis_name="s"),
        scratch_types=[
            pltpu.VMEM((epw,), jnp.int32),
            pltpu.VMEM((epw,), jnp.int32),
            pltpu.VMEM((bb, TW), jnp.int32),
            pltpu.VMEM((bb, TW), jnp.int32),
            pltpu.VMEM((bb, TW), jnp.int32),
            pltpu.VMEM((bb, TW), jnp.int32),
            pltpu.SemaphoreType.DMA,
            pltpu.SemaphoreType.DMA,
        ],
    )(p_tab, q_tab, ridx, cidx)


def _scatter(e2, row3d, ne, bb):
    epw = ne // NW
    nblk = epw // bb
    eps = ne // NC

    def body(e2, row3d, parts, idx2d, buf, buf1, zbuf, acc, sem0, sem1):
        cid = lax.axis_index("c")
        sid = lax.axis_index("s")
        w = cid * NS + sid
        pltpu.sync_copy(row3d.at[w], idx2d)

        def zrow(r, c):
            for j in range(LG):
                zbuf[r, pl.ds(j * 16, 16)] = jnp.zeros((16,), jnp.float32)
            return c

        lax.fori_loop(0, ZCH, zrow, 0)

        tbase = cid * eps + sid * epw

        def fire(b, bf, sem, t):
            pltpu.async_copy(e2.at[t, pl.ds(tbase + b * bb, bb)], bf, sem)

        def drain(bf, sem, t):
            pltpu.make_async_copy(e2.at[t, pl.ds(tbase, bb)], bf, sem).wait()

        for t in range(T):
            @pl.when(sid < ZTW)
            def _zero():
                def zc(k, c):
                    pltpu.sync_copy(zbuf,
                                    acc.at[pl.ds(sid * ZPT + k * ZCH, ZCH)])
                    return c
                lax.fori_loop(0, ZPT // ZCH, zc, 0)

            plsc.subcore_barrier()
            fire(0, buf, sem0, t)

            def pair(k, c):
                b0 = 2 * k
                fire(b0 + 1, buf1, sem1, t)
                drain(buf, sem0, t)
                pltpu.sync_copy(buf, acc.at[idx2d.at[b0]], add=True)
                fire(b0 + 2, buf, sem0, t)
                drain(buf1, sem1, t)
                pltpu.sync_copy(buf1, acc.at[idx2d.at[b0 + 1]], add=True)
                return c

            lax.fori_loop(0, (nblk - 1) // 2, pair, 0)
            drain(buf, sem0, t)
            pltpu.sync_copy(buf, acc.at[idx2d.at[nblk - 1]], add=True)
            plsc.subcore_barrier()

            @pl.when(sid < ZTW)
            def _readback():
                def rb(k, c):
                    off = sid * ZPT + k * ZCH
                    pltpu.sync_copy(acc.at[pl.ds(off, ZCH)],
                                    parts.at[t, cid, pl.ds(off, ZCH)])
                    return c
                lax.fori_loop(0, ZPT // ZCH, rb, 0)

            plsc.subcore_barrier()

    return pl.kernel(
        body,
        out_type=jax.ShapeDtypeStruct((T, NC, N, H), jnp.float32),
        mesh=plsc.VectorSubcoreMesh(core_axis_name="c", subcore_axis_name="s"),
        scratch_types=[
            pltpu.VMEM((nblk, bb), jnp.int32),
            pltpu.VMEM((bb, H), jnp.float32),
            pltpu.VMEM((bb, H), jnp.float32),
            pltpu.VMEM((ZCH, H), jnp.float32),
            pltpu.VMEM_SHARED((N, H), jnp.float32),
            pltpu.SemaphoreType.DMA,
            pltpu.SemaphoreType.DMA,
        ],
    )(e2, row3d)


BA = 2000


def _pack_bf16_pairs(x):
    xi = lax.bitcast_convert_type(x, jnp.int32)
    rne = xi + 0x7FFF + (lax.shift_right_logical(xi, 16) & 1)
    xb = lax.shift_right_logical(rne, 16)
    return xb[:, :W32] | (xb[:, W32:] << 16)


def _tables_tc(h_ref, wa_ref, wb_ref, p_ref, q_ref):
    for t in range(T):
        hb = h_ref[t]
        pf = jnp.dot(hb, wa_ref[...], preferred_element_type=jnp.float32)
        qf = jnp.dot(hb, wb_ref[...], preferred_element_type=jnp.float32)
        p_ref[:, t * W32:(t + 1) * W32] = _pack_bf16_pairs(pf)
        q_ref[:, t * W32:(t + 1) * W32] = _pack_bf16_pairs(qf)


def _tables(h, wa, wb):
    return pl.pallas_call(
        _tables_tc,
        grid=(N // BA,),
        in_specs=[
            pl.BlockSpec((T, BA, D), lambda j: (0, j, 0)),
            pl.BlockSpec((D, H), lambda j: (0, 0)),
            pl.BlockSpec((D, H), lambda j: (0, 0)),
        ],
        out_specs=[pl.BlockSpec((BA, TW), lambda j: (j, 0))] * 2,
        out_shape=[jax.ShapeDtypeStruct((N, TW), jnp.int32)] * 2,
    )(h, wa, wb)


BC = 2000


def _unpack_sum(pw, qw):
    plo = lax.bitcast_convert_type(pw << 16, jnp.float32)
    phi = lax.bitcast_convert_type(pw & jnp.int32(-65536), jnp.float32)
    qlo = lax.bitcast_convert_type(qw << 16, jnp.float32)
    qhi = lax.bitcast_convert_type(qw & jnp.int32(-65536), jnp.float32)
    return jnp.concatenate([plo + qlo, phi + qhi], axis=1)


def _edge_tc(p_ref, q_ref, rad_ref, wc_ref, b1_ref, w2_ref, b2_ref, e2_ref):
    p = p_ref[...]
    q = q_ref[...]
    for t in range(T):
        g = _unpack_sum(p[:, t * W32:(t + 1) * W32],
                        q[:, t * W32:(t + 1) * W32])
        e1 = (g
              + jnp.dot(rad_ref[t].astype(jnp.bfloat16),
                        wc_ref[...].astype(jnp.bfloat16),
                        preferred_element_type=jnp.float32)
              + b1_ref[...])
        e1 = jnp.maximum(e1, 0.0)
        e2 = jnp.dot(e1.astype(jnp.bfloat16), w2_ref[...].astype(jnp.bfloat16),
                     preferred_element_type=jnp.float32)
        e2_ref[t] = jnp.maximum(e2 + b2_ref[...], 0.0)


def _edge_mlp(p_rows, q_rows, radial, wc, b1, w2, b2, ne, off):
    return pl.pallas_call(
        _edge_tc,
        grid=(ne // BC,),
        in_specs=[
            pl.BlockSpec((BC, TW), lambda i: (i, 0)),
            pl.BlockSpec((BC, TW), lambda i: (i, 0)),
            pl.BlockSpec((T, BC, R), lambda i, off=off: (0, i + off, 0)),
            pl.BlockSpec((R, H), lambda i: (0, 0)),
            pl.BlockSpec((1, H), lambda i: (0, 0)),
            pl.BlockSpec((H, H), lambda i: (0, 0)),
            pl.BlockSpec((1, H), lambda i: (0, 0)),
        ],
        out_specs=pl.BlockSpec((T, BC, H), lambda i: (0, i, 0)),
        out_shape=jax.ShapeDtypeStruct((T, ne, H), jnp.float32),
    )(p_rows, q_rows, radial, wc, b1, w2, b2)


BN = 2000


def _node_tc(oth_ref, h_ref, pa_ref, wn1_ref, bn1_ref, wn2_ref,
             bn2_ref, hout_ref, a_ref):
    oth = oth_ref[0]
    hb = h_ref[0]
    agg = pa_ref[0, 0] + pa_ref[0, 1]
    a = jnp.concatenate([oth, hb, agg], axis=1)
    a_ref[0] = a
    z = jnp.maximum(
        jnp.dot(a.astype(jnp.bfloat16), wn1_ref[...].astype(jnp.bfloat16),
                preferred_element_type=jnp.float32)
        + bn1_ref[...], 0.0)
    o = jnp.dot(z.astype(jnp.bfloat16), wn2_ref[...].astype(jnp.bfloat16),
                preferred_element_type=jnp.float32) + bn2_ref[...]
    hout_ref[0] = hb + o


def _node_mlp(others, h, parts_a, wn1, bn1, wn2, bn2):
    return pl.pallas_call(
        _node_tc,
        grid=(T, N // BN),
        in_specs=[
            pl.BlockSpec((1, BN, H), lambda t, j: (t, j, 0)),
            pl.BlockSpec((1, BN, D), lambda t, j: (t, j, 0)),
            pl.BlockSpec((1, NC, BN, H), lambda t, j: (t, 0, j, 0)),
            pl.BlockSpec((H + D + H, H), lambda t, j: (0, 0)),
            pl.BlockSpec((1, H), lambda t, j: (0, 0)),
            pl.BlockSpec((H, D), lambda t, j: (0, 0)),
            pl.BlockSpec((1, D), lambda t, j: (0, 0)),
        ],
        out_specs=[
            pl.BlockSpec((1, BN, D), lambda t, j: (t, j, 0)),
            pl.BlockSpec((1, BN, H + D + H), lambda t, j: (t, j, 0)),
        ],
        out_shape=[
            jax.ShapeDtypeStruct((T, N, D), jnp.float32),
            jax.ShapeDtypeStruct((T, N, H + D + H), jnp.float32),
        ],
    )(others, h, parts_a, wn1, bn1, wn2, bn2)


def kernel(h, edge_index, radial, others,
           We1, be1, We2, be2, Wn1, bn1, Wn2, bn2):
    row = edge_index[0]
    col = edge_index[1]

    p32, q32 = _tables(h, We1[:D], We1[D:2 * D])
    pr, qr = _gather(p32, q32, row, col, E, BB)
    e2 = _edge_mlp(pr, qr, radial, We1[2 * D:], be1.reshape(1, H),
                   We2, be2.reshape(1, H), E, 0)
    parts = _scatter(e2, row.reshape(NW, NBLK, BB), E, BB)
    h_out, a_out = _node_mlp(others, h, parts, Wn1,
                             bn1.reshape(1, H), Wn2, bn2.reshape(1, D))
    return h_out, a_out


# speedup vs baseline: 1.0652x; 1.0108x over previous
import jax
import jax.numpy as jnp
from jax import lax
from jax.experimental import pallas as pl
from jax.experimental.pallas import tpu as pltpu
from jax.experimental.pallas import tpu_sc as plsc

T, N, E, D, H, R = 4, 10000, 320000, 128, 128, 16
NC, NS = 2, 16
NW = NC * NS
EPW = E // NW
BB = 80
NBLK = EPW // BB
EPS = E // NC
ZCH = 40
ZTW = 10
ZPT = N // ZTW
LG = H // 16
W32 = H // 2


TW = T * W32


def _gather(p_tab, q_tab, ridx, cidx, ne, bb):
    epw = ne // NW
    nblk = epw // bb

    def body(p_tab, q_tab, ridx, cidx, p_out, q_out,
             idx_r, idx_c, bp0, bq0, bp1, bq1, sem0, sem1):
        cid = lax.axis_index("c")
        sid = lax.axis_index("s")
        wid = sid * NC + cid

        def fire(bp, bq, sem, eb):
            pltpu.async_copy(p_tab.at[idx_r.at[pl.ds(eb, bb)]], bp, sem)
            pltpu.async_copy(q_tab.at[idx_c.at[pl.ds(eb, bb)]], bq, sem)

        def drain(bp, bq, sem):
            pltpu.make_async_copy(p_tab.at[idx_r.at[pl.ds(0, bb)]],
                                  bp, sem).wait()
            pltpu.make_async_copy(q_tab.at[idx_c.at[pl.ds(0, bb)]],
                                  bq, sem).wait()

        def store(bp, bq, base_out):
            pltpu.sync_copy(bp, p_out.at[pl.ds(base_out, bb)])
            pltpu.sync_copy(bq, q_out.at[pl.ds(base_out, bb)])

        base_e = wid * epw
        pltpu.sync_copy(ridx.at[pl.ds(base_e, epw)], idx_r)
        pltpu.sync_copy(cidx.at[pl.ds(base_e, epw)], idx_c)
        fire(bp0, bq0, sem0, 0)

        def pair(k, c):
            eb0 = 2 * k * bb
            fire(bp1, bq1, sem1, eb0 + bb)
            drain(bp0, bq0, sem0)
            store(bp0, bq0, base_e + eb0)
            fire(bp0, bq0, sem0, eb0 + 2 * bb)
            drain(bp1, bq1, sem1)
            store(bp1, bq1, base_e + eb0 + bb)
            return c

        lax.fori_loop(0, (nblk - 1) // 2, pair, 0)
        drain(bp0, bq0, sem0)
        store(bp0, bq0, base_e + (nblk - 1) * bb)

    return pl.kernel(
        body,
        out_type=(jax.ShapeDtypeStruct((ne, TW), jnp.int32),
                  jax.ShapeDtypeStruct((ne, TW), jnp.int32)),
        mesh=plsc.VectorSubcoreMesh(core_axis_name="c", subcore_axis_name="s"),
        scratch_types=[
            pltpu.VMEM((epw,), jnp.int32),
            pltpu.VMEM((epw,), jnp.int32),
            pltpu.VMEM((bb, TW), jnp.int32),
            pltpu.VMEM((bb, TW), jnp.int32),
            pltpu.VMEM((bb, TW), jnp.int32),
            pltpu.VMEM((bb, TW), jnp.int32),
            pltpu.SemaphoreType.DMA,
            pltpu.SemaphoreType.DMA,
        ],
    )(p_tab, q_tab, ridx, cidx)


def _scatter(e2, row3d, ne, bb):
    epw = ne // NW
    nblk = epw // bb
    eps = ne // NC

    def body(e2, row3d, parts, idx2d, buf, buf1, zbuf, acc, sem0, sem1):
        cid = lax.axis_index("c")
        sid = lax.axis_index("s")
        w = cid * NS + sid
        pltpu.sync_copy(row3d.at[w], idx2d)

        def zrow(r, c):
            for j in range(LG):
                zbuf[r, pl.ds(j * 16, 16)] = jnp.zeros((16,), jnp.float32)
            return c

        lax.fori_loop(0, ZCH, zrow, 0)

        tbase = cid * eps + sid * epw

        def fire(b, bf, sem, t):
            pltpu.async_copy(e2.at[t, pl.ds(tbase + b * bb, bb)], bf, sem)

        def drain(bf, sem, t):
            pltpu.make_async_copy(e2.at[t, pl.ds(tbase, bb)], bf, sem).wait()

        for t in range(T):
            @pl.when(sid < ZTW)
            def _zero():
                def zc(k, c):
                    pltpu.sync_copy(zbuf,
                                    acc.at[pl.ds(sid * ZPT + k * ZCH, ZCH)])
                    return c
                lax.fori_loop(0, ZPT // ZCH, zc, 0)

            plsc.subcore_barrier()
            fire(0, buf, sem0, t)

            def pair(k, c):
                b0 = 2 * k
                fire(b0 + 1, buf1, sem1, t)
                drain(buf, sem0, t)
                pltpu.sync_copy(buf, acc.at[idx2d.at[b0]], add=True)
                fire(b0 + 2, buf, sem0, t)
                drain(buf1, sem1, t)
                pltpu.sync_copy(buf1, acc.at[idx2d.at[b0 + 1]], add=True)
                return c

            lax.fori_loop(0, (nblk - 1) // 2, pair, 0)
            drain(buf, sem0, t)
            pltpu.sync_copy(buf, acc.at[idx2d.at[nblk - 1]], add=True)
            plsc.subcore_barrier()

            @pl.when(sid < ZTW)
            def _readback():
                def rb(k, c):
                    off = sid * ZPT + k * ZCH
                    pltpu.sync_copy(acc.at[pl.ds(off, ZCH)],
                                    parts.at[t, cid, pl.ds(off, ZCH)])
                    return c
                lax.fori_loop(0, ZPT // ZCH, rb, 0)

            plsc.subcore_barrier()

    return pl.kernel(
        body,
        out_type=jax.ShapeDtypeStruct((T, NC, N, H), jnp.float32),
        mesh=plsc.VectorSubcoreMesh(core_axis_name="c", subcore_axis_name="s"),
        scratch_types=[
            pltpu.VMEM((nblk, bb), jnp.int32),
            pltpu.VMEM((bb, H), jnp.float32),
            pltpu.VMEM((bb, H), jnp.float32),
            pltpu.VMEM((ZCH, H), jnp.float32),
            pltpu.VMEM_SHARED((N, H), jnp.float32),
            pltpu.SemaphoreType.DMA,
            pltpu.SemaphoreType.DMA,
        ],
    )(e2, row3d)


BA = 2000


def _pack_bf16_pairs(x):
    xi = lax.bitcast_convert_type(x, jnp.int32)
    rne = xi + 0x7FFF + (lax.shift_right_logical(xi, 16) & 1)
    xb = lax.shift_right_logical(rne, 16)
    return xb[:, :W32] | (xb[:, W32:] << 16)


def _tables_tc(h_ref, wa_ref, wb_ref, p_ref, q_ref):
    for t in range(T):
        hb = h_ref[t]
        pf = jnp.dot(hb, wa_ref[...], preferred_element_type=jnp.float32)
        qf = jnp.dot(hb, wb_ref[...], preferred_element_type=jnp.float32)
        p_ref[:, t * W32:(t + 1) * W32] = _pack_bf16_pairs(pf)
        q_ref[:, t * W32:(t + 1) * W32] = _pack_bf16_pairs(qf)


def _tables(h, wa, wb):
    return pl.pallas_call(
        _tables_tc,
        grid=(N // BA,),
        in_specs=[
            pl.BlockSpec((T, BA, D), lambda j: (0, j, 0)),
            pl.BlockSpec((D, H), lambda j: (0, 0)),
            pl.BlockSpec((D, H), lambda j: (0, 0)),
        ],
        out_specs=[pl.BlockSpec((BA, TW), lambda j: (j, 0))] * 2,
        out_shape=[jax.ShapeDtypeStruct((N, TW), jnp.int32)] * 2,
    )(h, wa, wb)


BC = 2000


def _unpack_sum(pw, qw):
    plo = lax.bitcast_convert_type(pw << 16, jnp.float32)
    phi = lax.bitcast_convert_type(pw & jnp.int32(-65536), jnp.float32)
    qlo = lax.bitcast_convert_type(qw << 16, jnp.float32)
    qhi = lax.bitcast_convert_type(qw & jnp.int32(-65536), jnp.float32)
    return jnp.concatenate([plo + qlo, phi + qhi], axis=1)


def _edge_tc(p_ref, q_ref, rad_ref, wc_ref, b1_ref, w2_ref, b2_ref, e2_ref):
    p = p_ref[...]
    q = q_ref[...]
    for t in range(T):
        g = _unpack_sum(p[:, t * W32:(t + 1) * W32],
                        q[:, t * W32:(t + 1) * W32])
        e1 = (g
              + jnp.dot(rad_ref[t], wc_ref[...],
                        preferred_element_type=jnp.float32)
              + b1_ref[...])
        e1 = jnp.maximum(e1, 0.0)
        e2 = jnp.dot(e1, w2_ref[...], preferred_element_type=jnp.float32)
        e2_ref[t] = jnp.maximum(e2 + b2_ref[...], 0.0)


def _edge_mlp(p_rows, q_rows, radial, wc, b1, w2, b2, ne, off):
    return pl.pallas_call(
        _edge_tc,
        grid=(ne // BC,),
        in_specs=[
            pl.BlockSpec((BC, TW), lambda i: (i, 0)),
            pl.BlockSpec((BC, TW), lambda i: (i, 0)),
            pl.BlockSpec((T, BC, R), lambda i, off=off: (0, i + off, 0)),
            pl.BlockSpec((R, H), lambda i: (0, 0)),
            pl.BlockSpec((1, H), lambda i: (0, 0)),
            pl.BlockSpec((H, H), lambda i: (0, 0)),
            pl.BlockSpec((1, H), lambda i: (0, 0)),
        ],
        out_specs=pl.BlockSpec((T, BC, H), lambda i: (0, i, 0)),
        out_shape=jax.ShapeDtypeStruct((T, ne, H), jnp.float32),
    )(p_rows, q_rows, radial, wc, b1, w2, b2)


BN = 2000


def _node_tc(oth_ref, h_ref, pa_ref, wn1_ref, bn1_ref, wn2_ref,
             bn2_ref, hout_ref, a_ref):
    oth = oth_ref[0]
    hb = h_ref[0]
    agg = pa_ref[0, 0] + pa_ref[0, 1]
    a = jnp.concatenate([oth, hb, agg], axis=1)
    a_ref[0] = a
    z = jnp.maximum(
        jnp.dot(a, wn1_ref[...], preferred_element_type=jnp.float32)
        + bn1_ref[...], 0.0)
    o = jnp.dot(z, wn2_ref[...], preferred_element_type=jnp.float32) \
        + bn2_ref[...]
    hout_ref[0] = hb + o


def _node_mlp(others, h, parts_a, wn1, bn1, wn2, bn2):
    return pl.pallas_call(
        _node_tc,
        grid=(T, N // BN),
        in_specs=[
            pl.BlockSpec((1, BN, H), lambda t, j: (t, j, 0)),
            pl.BlockSpec((1, BN, D), lambda t, j: (t, j, 0)),
            pl.BlockSpec((1, NC, BN, H), lambda t, j: (t, 0, j, 0)),
            pl.BlockSpec((H + D + H, H), lambda t, j: (0, 0)),
            pl.BlockSpec((1, H), lambda t, j: (0, 0)),
            pl.BlockSpec((H, D), lambda t, j: (0, 0)),
            pl.BlockSpec((1, D), lambda t, j: (0, 0)),
        ],
        out_specs=[
            pl.BlockSpec((1, BN, D), lambda t, j: (t, j, 0)),
            pl.BlockSpec((1, BN, H + D + H), lambda t, j: (t, j, 0)),
        ],
        out_shape=[
            jax.ShapeDtypeStruct((T, N, D), jnp.float32),
            jax.ShapeDtypeStruct((T, N, H + D + H), jnp.float32),
        ],
    )(others, h, parts_a, wn1, bn1, wn2, bn2)


def kernel(h, edge_index, radial, others,
           We1, be1, We2, be2, Wn1, bn1, Wn2, bn2):
    row = edge_index[0]
    col = edge_index[1]

    p32, q32 = _tables(h, We1[:D], We1[D:2 * D])
    pr, qr = _gather(p32, q32, row, col, E, BB)
    e2 = _edge_mlp(pr, qr, radial, We1[2 * D:], be1.reshape(1, H),
                   We2, be2.reshape(1, H), E, 0)
    parts = _scatter(e2, row.reshape(NW, NBLK, BB), E, BB)
    h_out, a_out = _node_mlp(others, h, parts, Wn1,
                             bn1.reshape(1, H), Wn2, bn2.reshape(1, D))
    return h_out, a_out
